# baseline (speedup 1.0000x reference)
import jax
import jax.numpy as jnp
from jax import lax
from jax.experimental import pallas as pl
from jax.experimental.pallas import tpu as pltpu
from jax.experimental.pallas import tpu_sc as plsc

BATCH = 16384
FACTOR = 16
VOCAB = 1000000
NUM_WORKERS = 32
PER_W = BATCH // NUM_WORKERS
CHUNKS = PER_W // 32
BLK = 512


def _body(user_hbm, item_hbm, avg_hbm, euT_hbm, eiT_hbm, ub_hbm, ib_hbm,
          out_hbm, idx_u, idx_i, u_blk, i_blk, bu, bi, avg_v, out_v,
          s0, s2, s3):
    wid = lax.axis_index("s") * 2 + lax.axis_index("c")
    base = wid * PER_W

    pltpu.sync_copy(user_hbm.at[pl.ds(base, PER_W)], idx_u)
    pltpu.sync_copy(item_hbm.at[pl.ds(base, PER_W)], idx_i)

    c_bu = pltpu.async_copy(ub_hbm.at[idx_u], bu, s2)
    c_bi = pltpu.async_copy(ib_hbm.at[idx_i], bi, s3)

    pltpu.sync_copy(avg_hbm.at[pl.ds(base, PER_W)], avg_v)
    c_bu.wait()
    c_bi.wait()

    lane = lax.iota(jnp.int32, 16)

    def step(c, carry):
        @pl.when(c < CHUNKS)
        def _fire():
            s = c * 32
            pb = lax.rem(c, 2) * BLK
            for half in range(2):
                u16 = idx_u[pl.ds(s + half * 16, 16)]
                v16 = idx_i[pl.ds(s + half * 16, 16)]
                for l in range(16):
                    u0 = (u16[l] // 16) * 16
                    v0 = (v16[l] // 16) * 16
                    dst = pb + (half * 16 + l) * 16
                    pltpu.async_copy(
                        euT_hbm.at[pl.ds(0, 2), pl.ds(0, 8), pl.ds(u0, 16)],
                        u_blk.at[pl.ds(0, 2), pl.ds(0, 8), pl.ds(dst, 16)],
                        s0)
                    pltpu.async_copy(
                        eiT_hbm.at[pl.ds(0, 2), pl.ds(0, 8), pl.ds(v0, 16)],
                        i_blk.at[pl.ds(0, 2), pl.ds(0, 8), pl.ds(dst, 16)],
                        s0)

        @pl.when(c > 0)
        def _drain_and_compute():
            cc = c - 1
            qb = lax.rem(cc, 2) * BLK
            dummy_src = euT_hbm.at[pl.ds(0, 2), pl.ds(0, 8), pl.ds(0, BLK)]
            for blk in (u_blk, i_blk):
                pltpu.make_async_copy(
                    dummy_src,
                    blk.at[pl.ds(0, 2), pl.ds(0, 8), pl.ds(0, BLK)],
                    s0).wait()

            for half in range(2):
                s = cc * 32 + half * 16
                u16 = idx_u[pl.ds(s, 16)]
                v16 = idx_i[pl.ds(s, 16)]
                ucol = qb + (half * 16 + lane) * 16 + (u16 & 15)
                vcol = qb + (half * 16 + lane) * 16 + (v16 & 15)
                acc = (avg_v[pl.ds(s, 16)] + bu[pl.ds(s, 16)]
                       + bi[pl.ds(s, 16)])
                for f in range(FACTOR):
                    fh = jnp.full((16,), f // 8, jnp.int32)
                    fr = jnp.full((16,), f % 8, jnp.int32)
                    uu = plsc.load_gather(u_blk, [fh, fr, ucol])
                    vv = plsc.load_gather(i_blk, [fh, fr, vcol])
                    acc = acc + uu * vv
                out_v[pl.ds(s, 16)] = acc

        return carry

    lax.fori_loop(0, CHUNKS + 1, step, 0)

    pltpu.sync_copy(out_v, out_hbm.at[pl.ds(base, PER_W)])


@jax.jit
def _cdcf(user, item, average, embed_user, embed_item, user_bias, item_bias):
    mesh = plsc.VectorSubcoreMesh(core_axis_name="c", subcore_axis_name="s")
    fn = pl.kernel(
        _body,
        out_type=jax.ShapeDtypeStruct((BATCH,), jnp.float32),
        mesh=mesh,
        scratch_types=[
            pltpu.VMEM((PER_W,), jnp.int32),
            pltpu.VMEM((PER_W,), jnp.int32),
            pltpu.VMEM((2, 8, 2 * BLK), jnp.float32),
            pltpu.VMEM((2, 8, 2 * BLK), jnp.float32),
            pltpu.VMEM((PER_W,), jnp.float32),
            pltpu.VMEM((PER_W,), jnp.float32),
            pltpu.VMEM((PER_W,), jnp.float32),
            pltpu.VMEM((PER_W,), jnp.float32),
            pltpu.SemaphoreType.DMA,
            pltpu.SemaphoreType.DMA,
            pltpu.SemaphoreType.DMA,
        ],
        compiler_params=pltpu.CompilerParams(needs_layout_passes=False,
                                             use_tc_tiling_on_sc=True,
                                             disable_bounds_checks=True,
                                             skip_device_barrier=True),
    )
    euT = embed_user.T.reshape(2, 8, VOCAB)
    eiT = embed_item.T.reshape(2, 8, VOCAB)
    return fn(user, item, average, euT, eiT, user_bias, item_bias)


def kernel(user, item, average, embed_user, embed_item, user_bias, item_bias):
    return _cdcf(user.astype(jnp.int32), item.astype(jnp.int32), average,
                 embed_user, embed_item, user_bias, item_bias)


# speedup vs baseline: 1.4998x; 1.4998x over previous
import jax
import jax.numpy as jnp
from jax import lax
from jax.experimental import pallas as pl
from jax.experimental.pallas import tpu as pltpu
from jax.experimental.pallas import tpu_sc as plsc

BATCH = 16384
FACTOR = 16
VOCAB = 1000000
NUM_WORKERS = 32
PER_W = BATCH // NUM_WORKERS
CHUNKS = PER_W // 16
BLK = 256


def _body(user_hbm, item_hbm, avg_hbm, euT_hbm, eiT_hbm, ub_hbm, ib_hbm,
          out_hbm, idx_u, idx_i, u_blk, i_blk, bu, bi, avg_v, out_v,
          s0, s2, s3):
    wid = lax.axis_index("s") * 2 + lax.axis_index("c")
    base = wid * PER_W

    pltpu.sync_copy(user_hbm.at[pl.ds(base, PER_W)], idx_u)
    pltpu.sync_copy(item_hbm.at[pl.ds(base, PER_W)], idx_i)

    c_bu = pltpu.async_copy(ub_hbm.at[idx_u], bu, s2)
    c_bi = pltpu.async_copy(ib_hbm.at[idx_i], bi, s3)

    pltpu.sync_copy(avg_hbm.at[pl.ds(base, PER_W)], avg_v)
    c_bu.wait()
    c_bi.wait()

    lane = lax.iota(jnp.int32, 16)

    def step(c, carry):
        @pl.when(c < CHUNKS)
        def _fire():
            s = c * 16
            pb = lax.rem(c, 2) * BLK
            u16 = idx_u[pl.ds(s, 16)]
            v16 = idx_i[pl.ds(s, 16)]
            for l in range(16):
                u0 = (u16[l] // 16) * 16
                v0 = (v16[l] // 16) * 16
                dst = pb + l * 16
                pltpu.async_copy(
                    euT_hbm.at[pl.ds(0, 2), pl.ds(0, 8), pl.ds(u0, 16)],
                    u_blk.at[pl.ds(0, 2), pl.ds(0, 8), pl.ds(dst, 16)], s0)
                pltpu.async_copy(
                    eiT_hbm.at[pl.ds(0, 2), pl.ds(0, 8), pl.ds(v0, 16)],
                    i_blk.at[pl.ds(0, 2), pl.ds(0, 8), pl.ds(dst, 16)], s0)

        @pl.when(c > 0)
        def _drain_and_compute():
            cc = c - 1
            s = cc * 16
            qb = lax.rem(cc, 2) * BLK
            dummy_src = euT_hbm.at[pl.ds(0, 2), pl.ds(0, 8), pl.ds(0, BLK)]
            for blk in (u_blk, i_blk):
                pltpu.make_async_copy(
                    dummy_src,
                    blk.at[pl.ds(0, 2), pl.ds(0, 8), pl.ds(0, BLK)],
                    s0).wait()

            u16 = idx_u[pl.ds(s, 16)]
            v16 = idx_i[pl.ds(s, 16)]
            ucol = qb + lane * 16 + (u16 & 15)
            vcol = qb + lane * 16 + (v16 & 15)
            acc = avg_v[pl.ds(s, 16)] + bu[pl.ds(s, 16)] + bi[pl.ds(s, 16)]
            for f in range(FACTOR):
                fh = jnp.full((16,), f // 8, jnp.int32)
                fr = jnp.full((16,), f % 8, jnp.int32)
                uu = plsc.load_gather(u_blk, [fh, fr, ucol])
                vv = plsc.load_gather(i_blk, [fh, fr, vcol])
                acc = acc + uu * vv
            out_v[pl.ds(s, 16)] = acc

        return carry

    lax.fori_loop(0, CHUNKS + 1, step, 0)

    pltpu.sync_copy(out_v, out_hbm.at[pl.ds(base, PER_W)])


@jax.jit
def _cdcf(user, item, average, embed_user, embed_item, user_bias, item_bias):
    mesh = plsc.VectorSubcoreMesh(core_axis_name="c", subcore_axis_name="s")
    fn = pl.kernel(
        _body,
        out_type=jax.ShapeDtypeStruct((BATCH,), jnp.float32),
        mesh=mesh,
        scratch_types=[
            pltpu.VMEM((PER_W,), jnp.int32),
            pltpu.VMEM((PER_W,), jnp.int32),
            pltpu.VMEM((2, 8, 2 * BLK), jnp.float32),
            pltpu.VMEM((2, 8, 2 * BLK), jnp.float32),
            pltpu.VMEM((PER_W,), jnp.float32),
            pltpu.VMEM((PER_W,), jnp.float32),
            pltpu.VMEM((PER_W,), jnp.float32),
            pltpu.VMEM((PER_W,), jnp.float32),
            pltpu.SemaphoreType.DMA,
            pltpu.SemaphoreType.DMA,
            pltpu.SemaphoreType.DMA,
        ],
        compiler_params=pltpu.CompilerParams(needs_layout_passes=False,
                                             use_tc_tiling_on_sc=True,
                                             disable_bounds_checks=True,
                                             skip_device_barrier=True),
    )
    euT = embed_user.T.reshape(2, 8, VOCAB)
    eiT = embed_item.T.reshape(2, 8, VOCAB)
    return fn(user, item, average, euT, eiT, user_bias, item_bias)


def kernel(user, item, average, embed_user, embed_item, user_bias, item_bias):
    return _cdcf(user.astype(jnp.int32), item.astype(jnp.int32), average,
                 embed_user, embed_item, user_bias, item_bias)


# speedup vs baseline: 1.6979x; 1.1321x over previous
import jax
import jax.numpy as jnp
from jax import lax
from jax.experimental import pallas as pl
from jax.experimental.pallas import tpu as pltpu
from jax.experimental.pallas import tpu_sc as plsc

BATCH = 16384
FACTOR = 16
VOCAB = 1000000
NUM_WORKERS = 32
PER_W = BATCH // NUM_WORKERS
CHUNKS = PER_W // 16
BLK = 256


def _body(user_hbm, item_hbm, avg_hbm, euT_hbm, eiT_hbm, ub_hbm, ib_hbm,
          out_hbm, idx_u, idx_i, u_blk, i_blk, bu, bi, avg_v, out_v,
          s0, s2, s3):
    wid = lax.axis_index("s") * 2 + lax.axis_index("c")
    base = wid * PER_W

    pltpu.sync_copy(user_hbm.at[pl.ds(base, PER_W)], idx_u)
    pltpu.sync_copy(item_hbm.at[pl.ds(base, PER_W)], idx_i)

    c_bu = pltpu.async_copy(ub_hbm.at[idx_u], bu, s2)
    c_bi = pltpu.async_copy(ib_hbm.at[idx_i], bi, s3)

    pltpu.sync_copy(avg_hbm.at[pl.ds(base, PER_W)], avg_v)
    c_bu.wait()
    c_bi.wait()

    lane = lax.iota(jnp.int32, 16)

    def step(c, carry):
        @pl.when(c < CHUNKS)
        def _fire():
            s = c * 16
            pb = lax.rem(c, 2) * BLK
            u16a = idx_u[pl.ds(s, 16)] & -16
            v16a = idx_i[pl.ds(s, 16)] & -16
            for l in range(16):
                u0 = pl.multiple_of(u16a[l], 16)
                v0 = pl.multiple_of(v16a[l], 16)
                dst = pb + l * 16
                pltpu.async_copy(
                    euT_hbm.at[pl.ds(0, 2), pl.ds(0, 8), pl.ds(u0, 16)],
                    u_blk.at[pl.ds(0, 2), pl.ds(0, 8), pl.ds(dst, 16)], s0)
                pltpu.async_copy(
                    eiT_hbm.at[pl.ds(0, 2), pl.ds(0, 8), pl.ds(v0, 16)],
                    i_blk.at[pl.ds(0, 2), pl.ds(0, 8), pl.ds(dst, 16)], s0)

        @pl.when(c > 0)
        def _drain_and_compute():
            cc = c - 1
            s = cc * 16
            qb = lax.rem(cc, 2) * BLK
            dummy_src = euT_hbm.at[pl.ds(0, 2), pl.ds(0, 8), pl.ds(0, BLK)]
            for blk in (u_blk, i_blk):
                pltpu.make_async_copy(
                    dummy_src,
                    blk.at[pl.ds(0, 2), pl.ds(0, 8), pl.ds(0, BLK)],
                    s0).wait()

            u16 = idx_u[pl.ds(s, 16)]
            v16 = idx_i[pl.ds(s, 16)]
            ucol = qb + lane * 16 + (u16 & 15)
            vcol = qb + lane * 16 + (v16 & 15)
            acc = avg_v[pl.ds(s, 16)] + bu[pl.ds(s, 16)] + bi[pl.ds(s, 16)]
            for f in range(FACTOR):
                fh = jnp.full((16,), f // 8, jnp.int32)
                fr = jnp.full((16,), f % 8, jnp.int32)
                uu = plsc.load_gather(u_blk, [fh, fr, ucol])
                vv = plsc.load_gather(i_blk, [fh, fr, vcol])
                acc = acc + uu * vv
            out_v[pl.ds(s, 16)] = acc

        return carry

    lax.fori_loop(0, CHUNKS + 1, step, 0)

    pltpu.sync_copy(out_v, out_hbm.at[pl.ds(base, PER_W)])


@jax.jit
def _cdcf(user, item, average, embed_user, embed_item, user_bias, item_bias):
    mesh = plsc.VectorSubcoreMesh(core_axis_name="c", subcore_axis_name="s")
    fn = pl.kernel(
        _body,
        out_type=jax.ShapeDtypeStruct((BATCH,), jnp.float32),
        mesh=mesh,
        scratch_types=[
            pltpu.VMEM((PER_W,), jnp.int32),
            pltpu.VMEM((PER_W,), jnp.int32),
            pltpu.VMEM((2, 8, 2 * BLK), jnp.float32),
            pltpu.VMEM((2, 8, 2 * BLK), jnp.float32),
            pltpu.VMEM((PER_W,), jnp.float32),
            pltpu.VMEM((PER_W,), jnp.float32),
            pltpu.VMEM((PER_W,), jnp.float32),
            pltpu.VMEM((PER_W,), jnp.float32),
            pltpu.SemaphoreType.DMA,
            pltpu.SemaphoreType.DMA,
            pltpu.SemaphoreType.DMA,
        ],
        compiler_params=pltpu.CompilerParams(needs_layout_passes=False,
                                             use_tc_tiling_on_sc=True,
                                             disable_bounds_checks=True,
                                             skip_device_barrier=True),
    )
    euT = embed_user.T.reshape(2, 8, VOCAB)
    eiT = embed_item.T.reshape(2, 8, VOCAB)
    return fn(user, item, average, euT, eiT, user_bias, item_bias)


def kernel(user, item, average, embed_user, embed_item, user_bias, item_bias):
    return _cdcf(user.astype(jnp.int32), item.astype(jnp.int32), average,
                 embed_user, embed_item, user_bias, item_bias)
